# baseline (speedup 1.0000x reference)
import functools

import jax
import jax.numpy as jnp
from jax import lax
from jax.experimental import pallas as pl
from jax.experimental.pallas import tpu as pltpu
from jax.experimental.pallas import tpu_sc as plsc


_TS = 512
_D = 2048
_E = 64
_N_TOK = 4 * 2048

_TAIL_TOK = 1536
_HEAD_TOK = _N_TOK - _TAIL_TOK

_NW = 32
_LANES = 16
_SC_CHUNK = 16384

_KS0 = 0
_KS1 = 42
_KS2 = _KS0 ^ _KS1 ^ 0x1BD11BDA
_ROTS = ((13, 15, 26, 6), (17, 29, 16, 24))
_ORDER = ((1, 2), (2, 0), (0, 1), (1, 2), (2, 0))
_KS = (_KS0, _KS1, _KS2)

_INJ0 = tuple(_KS[a] for a, _ in _ORDER)
_INJ1 = tuple((_KS[b] + i + 1) & 0xFFFFFFFF for i, (_, b) in enumerate(_ORDER))


def _rotl(x, r):
    return (x << jnp.uint32(r)) | (x >> jnp.uint32(32 - r))


def _noise_from_count(x1):
    x0 = x1
    x1 = _rotl(x1, _ROTS[0][0]) ^ x0
    first = True
    for i in range(5):
        for r in _ROTS[i % 2]:
            if first:
                first = False
                continue
            x0 = x0 + x1
            x1 = _rotl(x1, r)
            x1 = x1 ^ x0
        if _INJ0[i]:
            x0 = x0 + jnp.uint32(_INJ0[i])
        x1 = x1 + jnp.uint32(_INJ1[i])
    bits = x0 ^ x1
    u = lax.bitcast_convert_type(
        (bits >> jnp.uint32(9)) | jnp.uint32(0x3F800000), jnp.float32) - 1.0
    return u * jnp.float32(-0.02) + jnp.float32(1.01)


def _classify(new_attr, w_ref, b_ref, logits_ref, probs_ref, idx_ref):
    logits = lax.dot_general(
        new_attr, w_ref[...], (((1,), (0,)), ((), ())),
        preferred_element_type=jnp.float32) + b_ref[...]
    logits_ref[...] = logits
    m = jnp.max(logits, axis=-1, keepdims=True)
    e = jnp.exp(logits - m)
    probs = e / jnp.sum(e, axis=-1, keepdims=True)
    probs_ref[...] = probs
    idx_ref[0, 0, :] = jnp.argmax(probs, axis=-1).astype(jnp.int32)


def _head_kernel(hs_ref, w_ref, b_ref, logits_ref, probs_ref, idx_ref):
    t = pl.program_id(0)
    row = lax.broadcasted_iota(jnp.uint32, (_TS, _D), 0)
    col = lax.broadcasted_iota(jnp.uint32, (_TS, _D), 1)
    base = (t * (_TS * _D) + _KS1).astype(jnp.uint32)
    noise = _noise_from_count((row * jnp.uint32(_D) + col) + base)
    new_attr = hs_ref[...] * noise
    _classify(new_attr, w_ref, b_ref, logits_ref, probs_ref, idx_ref)


def _tail_kernel(na_ref, w_ref, b_ref, logits_ref, probs_ref, idx_ref):
    _classify(na_ref[...], w_ref, b_ref, logits_ref, probs_ref, idx_ref)


def _tc_call(body, first_arg, W, b2, n_tok):
    n_tiles = n_tok // _TS
    return pl.pallas_call(
        body,
        grid=(n_tiles,),
        in_specs=[
            pl.BlockSpec((_TS, _D), lambda t: (t, 0)),
            pl.BlockSpec((_D, _E), lambda t: (0, 0)),
            pl.BlockSpec((1, _E), lambda t: (0, 0)),
        ],
        out_specs=[
            pl.BlockSpec((_TS, _E), lambda t: (t, 0)),
            pl.BlockSpec((_TS, _E), lambda t: (t, 0)),
            pl.BlockSpec((1, 1, _TS), lambda t: (t, 0, 0)),
        ],
        out_shape=[
            jax.ShapeDtypeStruct((n_tok, _E), jnp.float32),
            jax.ShapeDtypeStruct((n_tok, _E), jnp.float32),
            jax.ShapeDtypeStruct((n_tiles, 1, _TS), jnp.int32),
        ],
    )(first_arg, W, b2)


_SC_TOTAL = _TAIL_TOK * _D
_SC_SPAN = _SC_TOTAL // _NW
_SC_BASE = _HEAD_TOK * _D


_SC_UNROLL = 8
_SC_NBUF = 3


def _sc_noise_kernel(hs_ref, out_ref, buf0, buf1, buf2, in_sem, out_sem):
    cid = lax.axis_index("c")
    sid = lax.axis_index("s")
    wid = cid * 16 + sid
    span_base = wid * _SC_SPAN
    lane = lax.iota(jnp.uint32, _LANES)
    bufs = (buf0, buf1, buf2)
    n_chunks = _SC_SPAN // _SC_CHUNK
    group = _LANES * _SC_UNROLL

    def fetch(ci):
        pltpu.async_copy(
            hs_ref.at[pl.ds(span_base + ci * _SC_CHUNK, _SC_CHUNK)],
            bufs[ci % _SC_NBUF], in_sem)

    def wait_in():
        pltpu.make_async_copy(
            hs_ref.at[pl.ds(span_base, _SC_CHUNK)], bufs[0], in_sem).wait()

    def wait_out():
        pltpu.make_async_copy(
            bufs[0], out_ref.at[pl.ds(span_base, _SC_CHUNK)], out_sem).wait()

    fetch(0)
    def chunk_body(ci, _):
        off = span_base + ci * _SC_CHUNK
        slot = lax.rem(ci, _SC_NBUF)
        wait_in()

        @pl.when(ci + 1 < n_chunks)
        def _():
            @pl.when(ci >= _SC_NBUF - 1)
            def _():
                wait_out()

            for s in range(_SC_NBUF):
                @pl.when(lax.rem(ci + 1, _SC_NBUF) == s)
                def _(s=s):
                    pltpu.async_copy(
                        hs_ref.at[pl.ds(off + _SC_CHUNK, _SC_CHUNK)],
                        bufs[s], in_sem)

        def vec_body(j, _):
            base_cnt = (jnp.uint32(_SC_BASE + _KS1)
                        + (off + j * group).astype(jnp.uint32))
            for s in range(_SC_NBUF):
                @pl.when(slot == s)
                def _(s=s):
                    b = bufs[s]
                    for k in range(_SC_UNROLL):
                        sl = pl.ds(j * group + k * _LANES, _LANES)
                        cnt = base_cnt + jnp.uint32(k * _LANES) + lane
                        b[sl] = b[sl] * _noise_from_count(cnt)
            return 0

        lax.fori_loop(0, _SC_CHUNK // group, vec_body, 0)

        for s in range(_SC_NBUF):
            @pl.when(slot == s)
            def _(s=s):
                pltpu.async_copy(
                    bufs[s], out_ref.at[pl.ds(off, _SC_CHUNK)], out_sem)
        return 0

    lax.fori_loop(0, n_chunks, chunk_body, 0)

    for _ in range(min(_SC_NBUF, n_chunks)):
        wait_out()


def _sc_noise(hs_tail_flat):
    mesh = plsc.VectorSubcoreMesh(core_axis_name="c", subcore_axis_name="s")
    run = functools.partial(
        pl.kernel,
        mesh=mesh,
        out_type=jax.ShapeDtypeStruct((_SC_TOTAL,), jnp.float32),
        scratch_types=[
            pltpu.VMEM((_SC_CHUNK,), jnp.float32),
            pltpu.VMEM((_SC_CHUNK,), jnp.float32),
            pltpu.VMEM((_SC_CHUNK,), jnp.float32),
            pltpu.SemaphoreType.DMA,
            pltpu.SemaphoreType.DMA,
        ],
    )(_sc_noise_kernel)
    return run(hs_tail_flat)


@jax.jit
def kernel(hidden_states, W, b):
    B, S, D = hidden_states.shape
    hs2 = hidden_states.reshape(B * S, D)
    b2 = b.reshape(1, _E)

    na_tail = _sc_noise(hs2[_HEAD_TOK:].reshape(-1)).reshape(_TAIL_TOK, D)

    logits_h, probs_h, idx_h = _tc_call(
        _head_kernel, hs2[:_HEAD_TOK], W, b2, _HEAD_TOK)

    logits_t, probs_t, idx_t = _tc_call(_tail_kernel, na_tail, W, b2, _TAIL_TOK)

    logits = jnp.concatenate([logits_h, logits_t], axis=0).reshape(B, S, _E)
    probs = jnp.concatenate([probs_h, probs_t], axis=0).reshape(B, S, _E)
    idx = jnp.concatenate([idx_h, idx_t], axis=0).reshape(B, S)
    return (idx, probs, logits)


# speedup vs baseline: 1.1727x; 1.1727x over previous
import functools

import jax
import jax.numpy as jnp
from jax import lax
from jax.experimental import pallas as pl
from jax.experimental.pallas import tpu as pltpu
from jax.experimental.pallas import tpu_sc as plsc


_TS = 512
_D = 2048
_E = 64
_N_TOK = 4 * 2048

_TAIL_TOK = 1536
_HEAD_TOK = _N_TOK - _TAIL_TOK

_NW = 32
_LANES = 16
_SC_CHUNK = 16384

_KS0 = 0
_KS1 = 42
_KS2 = _KS0 ^ _KS1 ^ 0x1BD11BDA
_ROTS = ((13, 15, 26, 6), (17, 29, 16, 24))
_ORDER = ((1, 2), (2, 0), (0, 1), (1, 2), (2, 0))
_KS = (_KS0, _KS1, _KS2)

_INJ0 = tuple(_KS[a] for a, _ in _ORDER)
_INJ1 = tuple((_KS[b] + i + 1) & 0xFFFFFFFF for i, (_, b) in enumerate(_ORDER))


def _rotl(x, r):
    return (x << jnp.uint32(r)) | (x >> jnp.uint32(32 - r))


def _noise_from_count(x1):
    x0 = x1
    x1 = _rotl(x1, _ROTS[0][0]) ^ x0
    first = True
    for i in range(5):
        for r in _ROTS[i % 2]:
            if first:
                first = False
                continue
            x0 = x0 + x1
            x1 = _rotl(x1, r)
            x1 = x1 ^ x0
        if _INJ0[i]:
            x0 = x0 + jnp.uint32(_INJ0[i])
        x1 = x1 + jnp.uint32(_INJ1[i])
    bits = x0 ^ x1
    u = lax.bitcast_convert_type(
        (bits >> jnp.uint32(9)) | jnp.uint32(0x3F800000), jnp.float32) - 1.0
    return u * jnp.float32(-0.02) + jnp.float32(1.01)


def _classify(new_attr, w_ref, b_ref, logits_ref, probs_ref, idx_ref):
    logits = lax.dot_general(
        new_attr, w_ref[...], (((1,), (0,)), ((), ())),
        preferred_element_type=jnp.float32) + b_ref[...]
    logits_ref[...] = logits
    m = jnp.max(logits, axis=-1, keepdims=True)
    e = jnp.exp(logits - m)
    probs = e / jnp.sum(e, axis=-1, keepdims=True)
    probs_ref[...] = probs
    idx_ref[0, 0, :] = jnp.argmax(probs, axis=-1).astype(jnp.int32)


def _head_kernel(hs_ref, w_ref, b_ref, logits_ref, probs_ref, idx_ref):
    t = pl.program_id(0)
    row = lax.broadcasted_iota(jnp.uint32, (_TS, _D), 0)
    col = lax.broadcasted_iota(jnp.uint32, (_TS, _D), 1)
    base = (t * (_TS * _D) + _KS1).astype(jnp.uint32)
    noise = _noise_from_count((row * jnp.uint32(_D) + col) + base)
    new_attr = hs_ref[...] * noise
    _classify(new_attr, w_ref, b_ref, logits_ref, probs_ref, idx_ref)


def _tail_kernel(na_ref, w_ref, b_ref, logits_ref, probs_ref, idx_ref):
    _classify(na_ref[...].reshape(_TS, _D), w_ref, b_ref,
              logits_ref, probs_ref, idx_ref)


def _tc_call(body, first_arg, W, b2, n_tok, flat_input=False):
    n_tiles = n_tok // _TS
    first_spec = (pl.BlockSpec((_TS * _D,), lambda t: (t,)) if flat_input
                  else pl.BlockSpec((_TS, _D), lambda t: (t, 0)))
    return pl.pallas_call(
        body,
        grid=(n_tiles,),
        in_specs=[
            first_spec,
            pl.BlockSpec((_D, _E), lambda t: (0, 0)),
            pl.BlockSpec((1, _E), lambda t: (0, 0)),
        ],
        out_specs=[
            pl.BlockSpec((_TS, _E), lambda t: (t, 0)),
            pl.BlockSpec((_TS, _E), lambda t: (t, 0)),
            pl.BlockSpec((1, 1, _TS), lambda t: (t, 0, 0)),
        ],
        out_shape=[
            jax.ShapeDtypeStruct((n_tok, _E), jnp.float32),
            jax.ShapeDtypeStruct((n_tok, _E), jnp.float32),
            jax.ShapeDtypeStruct((n_tiles, 1, _TS), jnp.int32),
        ],
    )(first_arg, W, b2)


_SC_TOTAL = _TAIL_TOK * _D
_SC_SPAN = _SC_TOTAL // _NW
_SC_BASE = _HEAD_TOK * _D


_SC_UNROLL = 8
_SC_NBUF = 3


def _sc_noise_kernel(hs_ref, out_ref, buf0, buf1, buf2, in_sem, out_sem):
    cid = lax.axis_index("c")
    sid = lax.axis_index("s")
    wid = cid * 16 + sid
    span_base = wid * _SC_SPAN
    lane = lax.iota(jnp.uint32, _LANES)
    bufs = (buf0, buf1, buf2)
    n_chunks = _SC_SPAN // _SC_CHUNK
    group = _LANES * _SC_UNROLL

    def fetch(ci):
        pltpu.async_copy(
            hs_ref.at[pl.ds(span_base + ci * _SC_CHUNK, _SC_CHUNK)],
            bufs[ci % _SC_NBUF], in_sem)

    def wait_in():
        pltpu.make_async_copy(
            hs_ref.at[pl.ds(span_base, _SC_CHUNK)], bufs[0], in_sem).wait()

    def wait_out():
        pltpu.make_async_copy(
            bufs[0], out_ref.at[pl.ds(span_base, _SC_CHUNK)], out_sem).wait()

    fetch(0)
    def chunk_body(ci, _):
        off = span_base + ci * _SC_CHUNK
        slot = lax.rem(ci, _SC_NBUF)
        wait_in()

        @pl.when(ci + 1 < n_chunks)
        def _():
            @pl.when(ci >= _SC_NBUF - 1)
            def _():
                wait_out()

            for s in range(_SC_NBUF):
                @pl.when(lax.rem(ci + 1, _SC_NBUF) == s)
                def _(s=s):
                    pltpu.async_copy(
                        hs_ref.at[pl.ds(off + _SC_CHUNK, _SC_CHUNK)],
                        bufs[s], in_sem)

        def vec_body(j, _):
            base_cnt = (jnp.uint32(_SC_BASE + _KS1)
                        + (off + j * group).astype(jnp.uint32))
            for s in range(_SC_NBUF):
                @pl.when(slot == s)
                def _(s=s):
                    b = bufs[s]
                    for k in range(_SC_UNROLL):
                        sl = pl.ds(j * group + k * _LANES, _LANES)
                        cnt = base_cnt + jnp.uint32(k * _LANES) + lane
                        b[sl] = b[sl] * _noise_from_count(cnt)
            return 0

        lax.fori_loop(0, _SC_CHUNK // group, vec_body, 0)

        for s in range(_SC_NBUF):
            @pl.when(slot == s)
            def _(s=s):
                pltpu.async_copy(
                    bufs[s], out_ref.at[pl.ds(off, _SC_CHUNK)], out_sem)
        return 0

    lax.fori_loop(0, n_chunks, chunk_body, 0)

    for _ in range(min(_SC_NBUF, n_chunks)):
        wait_out()


def _sc_noise(hs_tail_flat):
    mesh = plsc.VectorSubcoreMesh(core_axis_name="c", subcore_axis_name="s")
    run = functools.partial(
        pl.kernel,
        mesh=mesh,
        out_type=jax.ShapeDtypeStruct((_SC_TOTAL,), jnp.float32),
        scratch_types=[
            pltpu.VMEM((_SC_CHUNK,), jnp.float32),
            pltpu.VMEM((_SC_CHUNK,), jnp.float32),
            pltpu.VMEM((_SC_CHUNK,), jnp.float32),
            pltpu.SemaphoreType.DMA,
            pltpu.SemaphoreType.DMA,
        ],
    )(_sc_noise_kernel)
    return run(hs_tail_flat)


@jax.jit
def kernel(hidden_states, W, b):
    B, S, D = hidden_states.shape
    hs2 = hidden_states.reshape(B * S, D)
    b2 = b.reshape(1, _E)

    na_flat = _sc_noise(hs2[_HEAD_TOK:].reshape(-1))

    logits_h, probs_h, idx_h = _tc_call(_head_kernel, hs2, W, b2, _HEAD_TOK)

    logits_t, probs_t, idx_t = _tc_call(
        _tail_kernel, na_flat, W, b2, _TAIL_TOK, flat_input=True)

    logits = jnp.concatenate([logits_h, logits_t], axis=0).reshape(B, S, _E)
    probs = jnp.concatenate([probs_h, probs_t], axis=0).reshape(B, S, _E)
    idx = jnp.concatenate([idx_h, idx_t], axis=0).reshape(B, S)
    return (idx, probs, logits)


# speedup vs baseline: 1.2186x; 1.0392x over previous
import functools

import jax
import jax.numpy as jnp
from jax import lax
from jax.experimental import pallas as pl
from jax.experimental.pallas import tpu as pltpu
from jax.experimental.pallas import tpu_sc as plsc


_TS = 512
_D = 2048
_E = 64
_N_TOK = 4 * 2048

_TAIL_TOK = 2048
_HEAD_TOK = _N_TOK - _TAIL_TOK

_NW = 32
_LANES = 16
_SC_CHUNK = 16384

_KS0 = 0
_KS1 = 42
_KS2 = _KS0 ^ _KS1 ^ 0x1BD11BDA
_ROTS = ((13, 15, 26, 6), (17, 29, 16, 24))
_ORDER = ((1, 2), (2, 0), (0, 1), (1, 2), (2, 0))
_KS = (_KS0, _KS1, _KS2)

_INJ0 = tuple(_KS[a] for a, _ in _ORDER)
_INJ1 = tuple((_KS[b] + i + 1) & 0xFFFFFFFF for i, (_, b) in enumerate(_ORDER))


def _rotl(x, r):
    return (x << jnp.uint32(r)) | (x >> jnp.uint32(32 - r))


def _noise_from_count(x1):
    x0 = x1
    x1 = _rotl(x1, _ROTS[0][0]) ^ x0
    first = True
    for i in range(5):
        for r in _ROTS[i % 2]:
            if first:
                first = False
                continue
            x0 = x0 + x1
            x1 = _rotl(x1, r)
            x1 = x1 ^ x0
        if _INJ0[i]:
            x0 = x0 + jnp.uint32(_INJ0[i])
        x1 = x1 + jnp.uint32(_INJ1[i])
    bits = x0 ^ x1
    u = lax.bitcast_convert_type(
        (bits >> jnp.uint32(9)) | jnp.uint32(0x3F800000), jnp.float32) - 1.0
    return u * jnp.float32(-0.02) + jnp.float32(1.01)


def _classify(new_attr, w_ref, b_ref, logits_ref, probs_ref, idx_ref):
    logits = lax.dot_general(
        new_attr, w_ref[...], (((1,), (0,)), ((), ())),
        preferred_element_type=jnp.float32) + b_ref[...]
    logits_ref[...] = logits
    m = jnp.max(logits, axis=-1, keepdims=True)
    e = jnp.exp(logits - m)
    probs = e / jnp.sum(e, axis=-1, keepdims=True)
    probs_ref[...] = probs
    idx_ref[0, 0, :] = jnp.argmax(probs, axis=-1).astype(jnp.int32)


def _head_kernel(hs_ref, w_ref, b_ref, logits_ref, probs_ref, idx_ref):
    t = pl.program_id(0)
    row = lax.broadcasted_iota(jnp.uint32, (_TS, _D), 0)
    col = lax.broadcasted_iota(jnp.uint32, (_TS, _D), 1)
    base = (t * (_TS * _D) + _KS1).astype(jnp.uint32)
    noise = _noise_from_count((row * jnp.uint32(_D) + col) + base)
    new_attr = hs_ref[...] * noise
    _classify(new_attr, w_ref, b_ref, logits_ref, probs_ref, idx_ref)


def _tail_kernel(na_ref, w_ref, b_ref, logits_ref, probs_ref, idx_ref):
    _classify(na_ref[...].reshape(_TS, _D), w_ref, b_ref,
              logits_ref, probs_ref, idx_ref)


def _tc_call(body, first_arg, W, b2, n_tok, flat_input=False):
    n_tiles = n_tok // _TS
    first_spec = (pl.BlockSpec((_TS * _D,), lambda t: (t,)) if flat_input
                  else pl.BlockSpec((_TS, _D), lambda t: (t, 0)))
    return pl.pallas_call(
        body,
        grid=(n_tiles,),
        in_specs=[
            first_spec,
            pl.BlockSpec((_D, _E), lambda t: (0, 0)),
            pl.BlockSpec((1, _E), lambda t: (0, 0)),
        ],
        out_specs=[
            pl.BlockSpec((_TS, _E), lambda t: (t, 0)),
            pl.BlockSpec((_TS, _E), lambda t: (t, 0)),
            pl.BlockSpec((1, 1, _TS), lambda t: (t, 0, 0)),
        ],
        out_shape=[
            jax.ShapeDtypeStruct((n_tok, _E), jnp.float32),
            jax.ShapeDtypeStruct((n_tok, _E), jnp.float32),
            jax.ShapeDtypeStruct((n_tiles, 1, _TS), jnp.int32),
        ],
    )(first_arg, W, b2)


_SC_TOTAL = _TAIL_TOK * _D
_SC_SPAN = _SC_TOTAL // _NW
_SC_BASE = _HEAD_TOK * _D


_SC_UNROLL = 8
_SC_NBUF = 3


def _sc_noise_kernel(hs_ref, out_ref, buf0, buf1, buf2, in_sem, out_sem):
    cid = lax.axis_index("c")
    sid = lax.axis_index("s")
    wid = cid * 16 + sid
    span_base = wid * _SC_SPAN
    lane = lax.iota(jnp.uint32, _LANES)
    bufs = (buf0, buf1, buf2)
    n_chunks = _SC_SPAN // _SC_CHUNK
    group = _LANES * _SC_UNROLL

    def fetch(ci):
        pltpu.async_copy(
            hs_ref.at[pl.ds(span_base + ci * _SC_CHUNK, _SC_CHUNK)],
            bufs[ci % _SC_NBUF], in_sem)

    def wait_in():
        pltpu.make_async_copy(
            hs_ref.at[pl.ds(span_base, _SC_CHUNK)], bufs[0], in_sem).wait()

    def wait_out():
        pltpu.make_async_copy(
            bufs[0], out_ref.at[pl.ds(span_base, _SC_CHUNK)], out_sem).wait()

    fetch(0)
    def chunk_body(ci, _):
        off = span_base + ci * _SC_CHUNK
        slot = lax.rem(ci, _SC_NBUF)
        wait_in()

        @pl.when(ci + 1 < n_chunks)
        def _():
            @pl.when(ci >= _SC_NBUF - 1)
            def _():
                wait_out()

            for s in range(_SC_NBUF):
                @pl.when(lax.rem(ci + 1, _SC_NBUF) == s)
                def _(s=s):
                    pltpu.async_copy(
                        hs_ref.at[pl.ds(off + _SC_CHUNK, _SC_CHUNK)],
                        bufs[s], in_sem)

        def vec_body(j, _):
            base_cnt = (jnp.uint32(_SC_BASE + _KS1)
                        + (off + j * group).astype(jnp.uint32))
            for s in range(_SC_NBUF):
                @pl.when(slot == s)
                def _(s=s):
                    b = bufs[s]
                    for k in range(_SC_UNROLL):
                        sl = pl.ds(j * group + k * _LANES, _LANES)
                        cnt = base_cnt + jnp.uint32(k * _LANES) + lane
                        b[sl] = b[sl] * _noise_from_count(cnt)
            return 0

        lax.fori_loop(0, _SC_CHUNK // group, vec_body, 0)

        for s in range(_SC_NBUF):
            @pl.when(slot == s)
            def _(s=s):
                pltpu.async_copy(
                    bufs[s], out_ref.at[pl.ds(off, _SC_CHUNK)], out_sem)
        return 0

    lax.fori_loop(0, n_chunks, chunk_body, 0)

    for _ in range(min(_SC_NBUF, n_chunks)):
        wait_out()


def _sc_noise(hs_tail_flat):
    mesh = plsc.VectorSubcoreMesh(core_axis_name="c", subcore_axis_name="s")
    run = functools.partial(
        pl.kernel,
        mesh=mesh,
        out_type=jax.ShapeDtypeStruct((_SC_TOTAL,), jnp.float32),
        scratch_types=[
            pltpu.VMEM((_SC_CHUNK,), jnp.float32),
            pltpu.VMEM((_SC_CHUNK,), jnp.float32),
            pltpu.VMEM((_SC_CHUNK,), jnp.float32),
            pltpu.SemaphoreType.DMA,
            pltpu.SemaphoreType.DMA,
        ],
    )(_sc_noise_kernel)
    return run(hs_tail_flat)


@jax.jit
def kernel(hidden_states, W, b):
    B, S, D = hidden_states.shape
    hs2 = hidden_states.reshape(B * S, D)
    b2 = b.reshape(1, _E)

    na_flat = _sc_noise(hs2[_HEAD_TOK:].reshape(-1))

    logits_h, probs_h, idx_h = _tc_call(_head_kernel, hs2, W, b2, _HEAD_TOK)

    logits_t, probs_t, idx_t = _tc_call(
        _tail_kernel, na_flat, W, b2, _TAIL_TOK, flat_input=True)

    logits = jnp.concatenate([logits_h, logits_t], axis=0).reshape(B, S, _E)
    probs = jnp.concatenate([probs_h, probs_t], axis=0).reshape(B, S, _E)
    idx = jnp.concatenate([idx_h, idx_t], axis=0).reshape(B, S)
    return (idx, probs, logits)
